# baseline (speedup 1.0000x reference)
import jax
import jax.numpy as jnp
from jax import lax
from jax.experimental import pallas as pl
from jax.experimental.pallas import tpu as pltpu

K = 1024
D = 256
BETA = 0.25
HW = 1024
NB_PAIR = 4
NH = 1
H = HW // NH


def _vq_body(z_ref, cb2_ref, csq_ref, prob_ref, zq_ref, part_ref):
    cb2 = cb2_ref[...]
    csq = csq_ref[...]
    zblk = z_ref[...]

    probs = []
    logps = []
    sse = jnp.float32(0.0)
    for p in range(2):
        zt = zblk[p, 0].T
        zsq = jnp.sum(zt * zt, axis=1, keepdims=True)
        prod2 = lax.dot_general(zt, cb2, (((1,), (1,)), ((), ())),
                                preferred_element_type=jnp.float32)
        dist = (zsq + csq) - prod2
        dmin = jnp.min(dist, axis=1, keepdims=True)

        lane_iota = lax.broadcasted_iota(jnp.int32, (H, K), 1)
        idx = jnp.min(jnp.where(dist == dmin, lane_iota, K), axis=1)
        onehot = (lane_iota == idx[:, None]).astype(jnp.float32)
        zq = 0.5 * lax.dot_general(onehot, cb2, (((1,), (0,)), ((), ())),
                                   preferred_element_type=jnp.float32)
        diff = zq - zt
        sse = sse + jnp.sum(diff * diff)
        zq_ref[p, 0, :, :] = (zt + diff).T

        t = dmin - dist
        e = jnp.exp(t)
        zsum = jnp.sum(e, axis=1, keepdims=True)
        parr = e * (1.0 / zsum)
        logp = t - jnp.log(zsum)
        prob_ref[p, 0, :, :] = parr
        probs.append(parr)
        logps.append(logp)

    p0, p1 = probs
    lp0, lp1 = logps
    m = 0.5 * (p0 + p1)
    logm = jnp.log(m)
    tkl = p0 * (lp0 - logm) + p1 * (lp1 - logm)
    klsum = 0.5 * jnp.sum(tkl)
    went = p0 * lp0 + p1 * lp1
    hsum = -0.5 * jnp.sum(went)

    lane = lax.broadcasted_iota(jnp.int32, (1, 128), 1)
    vec = (jnp.where(lane == 0, sse, 0.0)
           + jnp.where(lane == 1, klsum, 0.0)
           + jnp.where(lane == 2, hsum, 0.0))
    part_ref[...] = vec.reshape(1, 1, 1, 128)


def kernel(z, i, it, codebook):
    b, d, h, w = z.shape
    del i, it
    zr = z.reshape(2, NB_PAIR, D, HW)
    csq = jnp.sum(codebook ** 2, axis=1).reshape(1, K)
    cb2 = codebook + codebook

    prob4, zq4, part = pl.pallas_call(
        _vq_body,
        grid=(NB_PAIR, NH),
        in_specs=[
            pl.BlockSpec((2, 1, D, H), lambda q, hh: (0, q, 0, hh)),
            pl.BlockSpec((K, D), lambda q, hh: (0, 0)),
            pl.BlockSpec((1, K), lambda q, hh: (0, 0)),
        ],
        out_specs=[
            pl.BlockSpec((2, 1, H, K), lambda q, hh: (0, q, hh, 0)),
            pl.BlockSpec((2, 1, D, H), lambda q, hh: (0, q, 0, hh)),
            pl.BlockSpec((1, 1, 1, 128), lambda q, hh: (q, hh, 0, 0)),
        ],
        out_shape=[
            jax.ShapeDtypeStruct((2, NB_PAIR, HW, K), jnp.float32),
            jax.ShapeDtypeStruct((2, NB_PAIR, D, HW), jnp.float32),
            jax.ShapeDtypeStruct((NB_PAIR, NH, 1, 128), jnp.float32),
        ],
        compiler_params=pltpu.CompilerParams(
            dimension_semantics=("parallel", "parallel")),
    )(zr, cb2, csq)

    distance_prob = prob4.reshape(b * h * w, K)
    z_q_out = zq4.reshape(b, d, h, w)
    tot = jnp.sum(part, axis=(0, 1, 2))
    n_el = jnp.float32(b * d * h * w)
    cb_loss = tot[0] / n_el
    q_loss = cb_loss + jnp.float32(BETA) * cb_loss
    n_pair = jnp.float32(b * h * w // 2)
    jsd = tot[1] / n_pair
    ent = tot[2] / n_pair
    return (z_q_out, q_loss, jsd, ent, distance_prob)


# speedup vs baseline: 1.0423x; 1.0423x over previous
import jax
import jax.numpy as jnp
from jax import lax
from jax.experimental import pallas as pl
from jax.experimental.pallas import tpu as pltpu

K = 1024
D = 256
BETA = 0.25
HW = 1024
NB_PAIR = 4
NH = 2
H = HW // NH


def _vq_body(z_ref, cb2_ref, csq_ref, prob_ref, zq_ref, part_ref):
    cb2 = cb2_ref[...]
    csq = csq_ref[...]
    zblk = z_ref[...]

    probs = []
    logps = []
    sse = jnp.float32(0.0)
    for p in range(2):
        zt = zblk[p, 0].T
        zsq = jnp.sum(zt * zt, axis=1, keepdims=True)
        prod2 = lax.dot_general(zt, cb2, (((1,), (1,)), ((), ())),
                                preferred_element_type=jnp.float32)
        dist = (zsq + csq) - prod2
        dmin = jnp.min(dist, axis=1, keepdims=True)
        sse = sse + jnp.sum(dmin)

        lane_iota = lax.broadcasted_iota(jnp.int32, (H, K), 1)
        masked = jnp.where(dist == dmin, lane_iota, K)
        idx = jnp.min(masked, axis=1)
        onehot = (lane_iota == idx[:, None]).astype(jnp.float32)
        zq = 0.5 * lax.dot_general(onehot, cb2, (((1,), (0,)), ((), ())),
                                   preferred_element_type=jnp.float32)
        zq_ref[p, 0, :, :] = (zt + (zq - zt)).T

        t = dmin - dist
        e = jnp.exp(t)
        zsum = jnp.sum(e, axis=1, keepdims=True)
        parr = e * (1.0 / zsum)
        logp = t - jnp.log(zsum)
        prob_ref[p, 0, :, :] = parr
        probs.append(parr)
        logps.append(logp)

    p0, p1 = probs
    lp0, lp1 = logps
    m2 = p0 + p1
    logm = jnp.log(0.5 * m2)
    went = p0 * lp0 + p1 * lp1
    tkl = went - m2 * logm
    klsum = 0.5 * jnp.sum(tkl)
    hsum = -0.5 * jnp.sum(went)

    lane = lax.broadcasted_iota(jnp.int32, (1, 128), 1)
    vec = (jnp.where(lane == 0, sse, 0.0)
           + jnp.where(lane == 1, klsum, 0.0)
           + jnp.where(lane == 2, hsum, 0.0))
    part_ref[...] = vec.reshape(1, 1, 1, 128)


def kernel(z, i, it, codebook):
    b, d, h, w = z.shape
    del i, it
    zr = z.reshape(2, NB_PAIR, D, HW)
    csq = jnp.sum(codebook ** 2, axis=1).reshape(1, K)
    cb2 = codebook + codebook

    prob4, zq4, part = pl.pallas_call(
        _vq_body,
        grid=(NB_PAIR, NH),
        in_specs=[
            pl.BlockSpec((2, 1, D, H), lambda q, hh: (0, q, 0, hh)),
            pl.BlockSpec((K, D), lambda q, hh: (0, 0)),
            pl.BlockSpec((1, K), lambda q, hh: (0, 0)),
        ],
        out_specs=[
            pl.BlockSpec((2, 1, H, K), lambda q, hh: (0, q, hh, 0)),
            pl.BlockSpec((2, 1, D, H), lambda q, hh: (0, q, 0, hh)),
            pl.BlockSpec((1, 1, 1, 128), lambda q, hh: (q, hh, 0, 0)),
        ],
        out_shape=[
            jax.ShapeDtypeStruct((2, NB_PAIR, HW, K), jnp.float32),
            jax.ShapeDtypeStruct((2, NB_PAIR, D, HW), jnp.float32),
            jax.ShapeDtypeStruct((NB_PAIR, NH, 1, 128), jnp.float32),
        ],
        compiler_params=pltpu.CompilerParams(
            dimension_semantics=("parallel", "parallel")),
    )(zr, cb2, csq)

    distance_prob = prob4.reshape(b * h * w, K)
    z_q_out = zq4.reshape(b, d, h, w)
    tot = jnp.sum(part, axis=(0, 1, 2))
    n_el = jnp.float32(b * d * h * w)
    cb_loss = tot[0] / n_el
    q_loss = cb_loss + jnp.float32(BETA) * cb_loss
    n_pair = jnp.float32(b * h * w // 2)
    jsd = tot[1] / n_pair
    ent = tot[2] / n_pair
    return (z_q_out, q_loss, jsd, ent, distance_prob)


# speedup vs baseline: 1.0876x; 1.0435x over previous
import jax
import jax.numpy as jnp
from jax import lax
from jax.experimental import pallas as pl
from jax.experimental.pallas import tpu as pltpu

K = 1024
D = 256
BETA = 0.25
HW = 1024
NB_PAIR = 4
NH = 2
H = HW // NH


def _vq_body(z_ref, cb2_ref, csq_ref, prob_ref, zq_ref, part_ref):
    cb2 = cb2_ref[...]
    csq = csq_ref[...]
    zblk = z_ref[...]

    probs = []
    logps = []
    sse = jnp.float32(0.0)
    for p in range(2):
        zt = zblk[p, 0].T
        zsq = jnp.sum(zt * zt, axis=1, keepdims=True)
        prod2 = lax.dot_general(zt, cb2, (((1,), (1,)), ((), ())),
                                preferred_element_type=jnp.float32)
        dist = (zsq + csq) - prod2
        dmin = jnp.min(dist, axis=1, keepdims=True)
        sse = sse + jnp.sum(dmin)

        lane_iota = lax.broadcasted_iota(jnp.int32, (H, K), 1).astype(jnp.float32)
        masked = jnp.where(dist == dmin, lane_iota, jnp.float32(K))
        idxf = jnp.min(masked, axis=1, keepdims=True)
        onehot = (lane_iota == idxf).astype(jnp.float32)
        zq = 0.5 * lax.dot_general(onehot, cb2, (((1,), (0,)), ((), ())),
                                   preferred_element_type=jnp.float32)
        zq_ref[p, 0, :, :] = (zt + (zq - zt)).T

        t = dmin - dist
        e = jnp.exp(t)
        zsum = jnp.sum(e, axis=1, keepdims=True)
        parr = e * (1.0 / zsum)
        logp = t - jnp.log(zsum)
        prob_ref[p, 0, :, :] = parr
        probs.append(parr)
        logps.append(logp)

    p0, p1 = probs
    lp0, lp1 = logps
    m2 = p0 + p1
    logm = jnp.log(0.5 * m2)
    went = p0 * lp0 + p1 * lp1
    tkl = went - m2 * logm
    klsum = 0.5 * jnp.sum(tkl)
    hsum = -0.5 * jnp.sum(went)

    lane = lax.broadcasted_iota(jnp.int32, (1, 128), 1)
    vec = (jnp.where(lane == 0, sse, 0.0)
           + jnp.where(lane == 1, klsum, 0.0)
           + jnp.where(lane == 2, hsum, 0.0))
    part_ref[...] = vec.reshape(1, 1, 1, 128)


def kernel(z, i, it, codebook):
    b, d, h, w = z.shape
    del i, it
    zr = z.reshape(2, NB_PAIR, D, HW)
    csq = jnp.sum(codebook ** 2, axis=1).reshape(1, K)
    cb2 = codebook + codebook

    prob4, zq4, part = pl.pallas_call(
        _vq_body,
        grid=(NB_PAIR, NH),
        in_specs=[
            pl.BlockSpec((2, 1, D, H), lambda q, hh: (0, q, 0, hh)),
            pl.BlockSpec((K, D), lambda q, hh: (0, 0)),
            pl.BlockSpec((1, K), lambda q, hh: (0, 0)),
        ],
        out_specs=[
            pl.BlockSpec((2, 1, H, K), lambda q, hh: (0, q, hh, 0)),
            pl.BlockSpec((2, 1, D, H), lambda q, hh: (0, q, 0, hh)),
            pl.BlockSpec((1, 1, 1, 128), lambda q, hh: (q, hh, 0, 0)),
        ],
        out_shape=[
            jax.ShapeDtypeStruct((2, NB_PAIR, HW, K), jnp.float32),
            jax.ShapeDtypeStruct((2, NB_PAIR, D, HW), jnp.float32),
            jax.ShapeDtypeStruct((NB_PAIR, NH, 1, 128), jnp.float32),
        ],
        compiler_params=pltpu.CompilerParams(
            dimension_semantics=("parallel", "parallel")),
    )(zr, cb2, csq)

    distance_prob = prob4.reshape(b * h * w, K)
    z_q_out = zq4.reshape(b, d, h, w)
    tot = jnp.sum(part, axis=(0, 1, 2))
    n_el = jnp.float32(b * d * h * w)
    cb_loss = tot[0] / n_el
    q_loss = cb_loss + jnp.float32(BETA) * cb_loss
    n_pair = jnp.float32(b * h * w // 2)
    jsd = tot[1] / n_pair
    ent = tot[2] / n_pair
    return (z_q_out, q_loss, jsd, ent, distance_prob)
